# Initial kernel scaffold; baseline (speedup 1.0000x reference)
import jax
import jax.numpy as jnp
from jax.experimental import pallas as pl


def kernel(x, table):
    raise NotImplementedError("write your pallas kernel here")


# speedup vs baseline: 1.6661x; 1.6661x over previous
import jax
import jax.numpy as jnp
from jax.experimental import pallas as pl

_BT = 1024


def _add_kernel(x_ref, t_ref, o_ref):
    o_ref[...] = x_ref[...] + t_ref[...]


def kernel(x, table):
    B, T, D = x.shape
    grid = (T // _BT, B)
    return pl.pallas_call(
        _add_kernel,
        grid=grid,
        in_specs=[
            pl.BlockSpec((1, _BT, D), lambda i, b: (b, i, 0)),
            pl.BlockSpec((_BT, D), lambda i, b: (i, 0)),
        ],
        out_specs=pl.BlockSpec((1, _BT, D), lambda i, b: (b, i, 0)),
        out_shape=jax.ShapeDtypeStruct((B, T, D), x.dtype),
    )(x, table)


# speedup vs baseline: 1.7357x; 1.0418x over previous
import jax
import jax.numpy as jnp
from jax.experimental import pallas as pl

_BT = 2048


def _add_kernel(x_ref, t_ref, o_ref):
    o_ref[...] = x_ref[...] + t_ref[...]


def kernel(x, table):
    B, T, D = x.shape
    grid = (T // _BT, B)
    return pl.pallas_call(
        _add_kernel,
        grid=grid,
        in_specs=[
            pl.BlockSpec((1, _BT, D), lambda i, b: (b, i, 0)),
            pl.BlockSpec((_BT, D), lambda i, b: (i, 0)),
        ],
        out_specs=pl.BlockSpec((1, _BT, D), lambda i, b: (b, i, 0)),
        out_shape=jax.ShapeDtypeStruct((B, T, D), x.dtype),
    )(x, table)
